# Initial kernel scaffold; baseline (speedup 1.0000x reference)
import jax
import jax.numpy as jnp
from jax.experimental import pallas as pl


def kernel(xyz1, xyz2):
    raise NotImplementedError("write your pallas kernel here")


# speedup vs baseline: 1.0238x; 1.0238x over previous
import jax
import jax.numpy as jnp
from jax.experimental import pallas as pl
from jax.experimental.pallas import tpu as pltpu

_B, _N1, _N2, _D = 4, 4096, 4096, 3
_DP = 8
_BLK1 = 512
_NB1 = _N1 // _BLK1


def _chamfer_body(x1_ref, x2_ref, s1_ref, s2_ref, cm_ref):
    i = pl.program_id(1)
    a = x1_ref[0]
    b = x2_ref[0]
    inner = jax.lax.dot_general(
        a, b, (((0,), (0,)), ((), ())),
        preferred_element_type=jnp.float32)
    sq1 = jnp.sum(a * a, axis=0)[:, None]
    sq2 = jnp.sum(b * b, axis=0)[None, :]
    d = jnp.maximum(sq1 + sq2 - 2.0 * inner, 0.0)
    colmin = jnp.min(d, axis=0)[None, :]
    rowsum = jnp.sum(jnp.min(d, axis=1))

    @pl.when(i == 0)
    def _():
        s1_ref[...] = rowsum.reshape(1, 1, 1)
        cm_ref[...] = colmin

    @pl.when(i > 0)
    def _():
        s1_ref[...] += rowsum.reshape(1, 1, 1)
        cm_ref[...] = jnp.minimum(cm_ref[...], colmin)

    @pl.when(i == _NB1 - 1)
    def _():
        s2_ref[...] = jnp.sum(cm_ref[...]).reshape(1, 1, 1)


def kernel(xyz1, xyz2):
    x1t = jnp.pad(jnp.moveaxis(xyz1, -1, -2), ((0, 0), (0, _DP - _D), (0, 0)))
    x2t = jnp.pad(jnp.moveaxis(xyz2, -1, -2), ((0, 0), (0, _DP - _D), (0, 0)))

    s1, s2 = pl.pallas_call(
        _chamfer_body,
        grid=(_B, _NB1),
        in_specs=[
            pl.BlockSpec((1, _DP, _BLK1), lambda b, i: (b, 0, i)),
            pl.BlockSpec((1, _DP, _N2), lambda b, i: (b, 0, 0)),
        ],
        out_specs=[
            pl.BlockSpec((1, 1, 1), lambda b, i: (b, 0, 0)),
            pl.BlockSpec((1, 1, 1), lambda b, i: (b, 0, 0)),
        ],
        out_shape=[
            jax.ShapeDtypeStruct((_B, 1, 1), jnp.float32),
            jax.ShapeDtypeStruct((_B, 1, 1), jnp.float32),
        ],
        scratch_shapes=[pltpu.VMEM((1, _N2), jnp.float32)],
        compiler_params=pltpu.CompilerParams(
            dimension_semantics=("parallel", "arbitrary")),
    )(x1t, x2t)

    return jnp.sum(s1) / (_B * _N1), jnp.sum(s2) / (_B * _N2)
